# baseline (speedup 1.0000x reference)
import functools

import jax
import jax.numpy as jnp
from jax import lax
from jax.experimental import pallas as pl
from jax.experimental.pallas import tpu as pltpu
from jax.experimental.pallas import tpu_sc as plsc

_NUM_BLUEPRINTS = 1_000_000
_NULL_IDX = _NUM_BLUEPRINTS
_D = 32
_L = 16
_BCHUNK = 128
_RING = 8
_AHEAD = 4
_NC = 2
_NS = 16
_NW = _NC * _NS


def _make_kernel(s, b):
    b_per_w = b // _NW
    chunks = b_per_w // _BCHUNK
    assert s > 2 * _RING
    rounds = (s - _AHEAD) // _RING
    tail0 = _AHEAD + _RING * rounds

    mesh = plsc.VectorSubcoreMesh(
        core_axis_name="c", subcore_axis_name="s",
        num_cores=_NC, num_subcores=_NS)

    @functools.partial(
        pl.kernel,
        out_type=jax.ShapeDtypeStruct((s, b, _D), jnp.float32),
        mesh=mesh,
        compiler_params=pltpu.CompilerParams(use_tc_tiling_on_sc=False),
        scratch_types=[
            pltpu.VMEM((2, s, _BCHUNK), jnp.int32),
            pltpu.VMEM((_RING, _BCHUNK, _D), jnp.float32),
        ]
        + [pltpu.SemaphoreType.DMA] * (2 * _RING),
    )
    def k(table_hbm, idx_hbm, out_hbm, idx_v, rows_v, *sems):
        gsems = sems[:_RING]
        ssems = sems[_RING:]
        wid = lax.axis_index("s") * _NC + lax.axis_index("c")

        def remap(islot):
            def body(j, carry):
                for l in range(_BCHUNK // _L):
                    v = idx_v[islot, j, pl.ds(l * _L, _L)]
                    v = jnp.where(v < 0, jnp.int32(_NULL_IDX), v)
                    idx_v[islot, j, pl.ds(l * _L, _L)] = v
                return carry
            lax.fori_loop(0, s, body, 0)

        def gather(islot, sidx, slot):
            return pltpu.async_copy(
                table_hbm.at[idx_v.at[islot, sidx]],
                rows_v.at[slot], gsems[slot])

        def g_wait(islot, sidx, slot):
            pltpu.make_async_copy(
                table_hbm.at[idx_v.at[islot, sidx]],
                rows_v.at[slot], gsems[slot]).wait()

        def store(sidx, b0, slot):
            return pltpu.async_copy(
                rows_v.at[slot],
                out_hbm.at[sidx, pl.ds(b0, _BCHUNK)],
                ssems[slot])

        def st_wait(sidx, b0, slot):
            pltpu.make_async_copy(
                rows_v.at[slot],
                out_hbm.at[sidx, pl.ds(b0, _BCHUNK)],
                ssems[slot]).wait()

        def step(t, slot, nslot, has_prev, islot, b0):
            g_wait(islot, t, slot)
            store(t, b0, slot)
            if has_prev:
                st_wait(t - (_RING - _AHEAD), b0, nslot)

                @pl.when(t + _AHEAD < s)
                def _():
                    gather(islot, t + _AHEAD, nslot)
            else:
                gather(islot, t + _AHEAD, nslot)

        for c in range(chunks):
            b0 = wid * b_per_w + c * _BCHUNK
            islot = c % 2
            pltpu.sync_copy(idx_hbm.at[:, pl.ds(b0, _BCHUNK)],
                            idx_v.at[islot])
            remap(islot)

            for u in range(_AHEAD):
                gather(islot, u, u)

            for t in range(_AHEAD):
                step(t, t % _RING, (t + _AHEAD) % _RING, False, islot, b0)

            def round_body(r, carry, islot=islot, b0=b0):
                base = _AHEAD + r * _RING
                for u in range(_RING):
                    slot = (_AHEAD + u) % _RING
                    nslot = (2 * _AHEAD + u) % _RING
                    step(base + u, slot, nslot, True, islot, b0)
                return carry

            lax.fori_loop(0, rounds, round_body, 0)

            for t in range(tail0, s):
                step(t, t % _RING, (t + _AHEAD) % _RING, True, islot, b0)
            for t in range(s - _AHEAD, s):
                st_wait(t, b0, t % _RING)

    return k


def kernel(blueprint_indices, embedding_weight):
    b, s = blueprint_indices.shape
    idx_t = jnp.transpose(blueprint_indices.astype(jnp.int32))
    out_t = _make_kernel(s, b)(embedding_weight, idx_t)
    return jnp.transpose(out_t, (1, 0, 2))


# speedup vs baseline: 1.0052x; 1.0052x over previous
import functools

import jax
import jax.numpy as jnp
from jax import lax
from jax.experimental import pallas as pl
from jax.experimental.pallas import tpu as pltpu
from jax.experimental.pallas import tpu_sc as plsc

_NUM_BLUEPRINTS = 1_000_000
_NULL_IDX = _NUM_BLUEPRINTS
_D = 32
_L = 16
_BCHUNK = 256
_RING = 6
_AHEAD = 3
_NC = 2
_NS = 16
_NW = _NC * _NS


def _make_kernel(s, b):
    b_per_w = b // _NW
    chunks = b_per_w // _BCHUNK
    assert s > 2 * _RING
    rounds = (s - _AHEAD) // _RING
    tail0 = _AHEAD + _RING * rounds

    mesh = plsc.VectorSubcoreMesh(
        core_axis_name="c", subcore_axis_name="s",
        num_cores=_NC, num_subcores=_NS)

    @functools.partial(
        pl.kernel,
        out_type=jax.ShapeDtypeStruct((s, b, _D), jnp.float32),
        mesh=mesh,
        compiler_params=pltpu.CompilerParams(use_tc_tiling_on_sc=False),
        scratch_types=[
            pltpu.VMEM((2, s, _BCHUNK), jnp.int32),
            pltpu.VMEM((_RING, _BCHUNK, _D), jnp.float32),
        ]
        + [pltpu.SemaphoreType.DMA] * (2 * _RING),
    )
    def k(table_hbm, idx_hbm, out_hbm, idx_v, rows_v, *sems):
        gsems = sems[:_RING]
        ssems = sems[_RING:]
        wid = lax.axis_index("s") * _NC + lax.axis_index("c")

        def remap(islot):
            def body(j, carry):
                for l in range(_BCHUNK // _L):
                    v = idx_v[islot, j, pl.ds(l * _L, _L)]
                    v = jnp.where(v < 0, jnp.int32(_NULL_IDX), v)
                    idx_v[islot, j, pl.ds(l * _L, _L)] = v
                return carry
            lax.fori_loop(0, s, body, 0)

        def gather(islot, sidx, slot):
            for h in range(2):
                pltpu.async_copy(
                    table_hbm.at[idx_v.at[islot, sidx, pl.ds(h * 128, 128)]],
                    rows_v.at[slot, pl.ds(h * 128, 128)], gsems[slot])

        def g_wait(islot, sidx, slot):
            for h in range(2):
                pltpu.make_async_copy(
                    table_hbm.at[idx_v.at[islot, sidx, pl.ds(h * 128, 128)]],
                    rows_v.at[slot, pl.ds(h * 128, 128)], gsems[slot]).wait()

        def store(sidx, b0, slot):
            return pltpu.async_copy(
                rows_v.at[slot],
                out_hbm.at[sidx, pl.ds(b0, _BCHUNK)],
                ssems[slot])

        def st_wait(sidx, b0, slot):
            pltpu.make_async_copy(
                rows_v.at[slot],
                out_hbm.at[sidx, pl.ds(b0, _BCHUNK)],
                ssems[slot]).wait()

        def step(t, slot, nslot, has_prev, islot, b0):
            g_wait(islot, t, slot)
            store(t, b0, slot)
            if has_prev:
                st_wait(t - (_RING - _AHEAD), b0, nslot)

                @pl.when(t + _AHEAD < s)
                def _():
                    gather(islot, t + _AHEAD, nslot)
            else:
                gather(islot, t + _AHEAD, nslot)

        for c in range(chunks):
            b0 = wid * b_per_w + c * _BCHUNK
            islot = c % 2
            pltpu.sync_copy(idx_hbm.at[:, pl.ds(b0, _BCHUNK)],
                            idx_v.at[islot])
            remap(islot)

            for u in range(_AHEAD):
                gather(islot, u, u)

            for t in range(_AHEAD):
                step(t, t % _RING, (t + _AHEAD) % _RING, False, islot, b0)

            def round_body(r, carry, islot=islot, b0=b0):
                base = _AHEAD + r * _RING
                for u in range(_RING):
                    slot = (_AHEAD + u) % _RING
                    nslot = (2 * _AHEAD + u) % _RING
                    step(base + u, slot, nslot, True, islot, b0)
                return carry

            lax.fori_loop(0, rounds, round_body, 0)

            for t in range(tail0, s):
                step(t, t % _RING, (t + _AHEAD) % _RING, True, islot, b0)
            for t in range(s - _AHEAD, s):
                st_wait(t, b0, t % _RING)

    return k


def kernel(blueprint_indices, embedding_weight):
    b, s = blueprint_indices.shape
    idx_t = jnp.transpose(blueprint_indices.astype(jnp.int32))
    out_t = _make_kernel(s, b)(embedding_weight, idx_t)
    return jnp.transpose(out_t, (1, 0, 2))
